# baseline (speedup 1.0000x reference)
import functools

import jax
import jax.numpy as jnp
from jax import lax
from jax.experimental import pallas as pl
from jax.experimental.pallas import tpu as pltpu
from jax.experimental.pallas import tpu_sc as plsc

D = 2048
H = 4096
E = 8
NT = 4096
EP = 128
T = 512
NBR = NT // T + E - 1
NPAD = NBR * T
HT = 1024
KH = H // HT
TS = 512
HTS = 512
KHS = H // HTS
SUB = 32

_INV_SQRT2 = 0.7071067811865476


def _gelu(z):
    return 0.5 * z * (1.0 + lax.erf(z * _INV_SQRT2))


def _router_body(x_ref, wg_ref, bg_ref, sig_ref, gw_ref, scale_ref, dest_ref,
                 counts_ref, aux_ref, oh_ref):
    logits = jnp.dot(x_ref[...], wg_ref[...],
                     preferred_element_type=jnp.float32) + bg_ref[...]
    col = lax.broadcasted_iota(jnp.int32, (NT, EP), 1)
    valid = col < E
    neglog = jnp.where(valid, logits, -jnp.inf)
    m = jnp.max(neglog, axis=1, keepdims=True)
    ex = jnp.where(valid, jnp.exp(neglog - m), 0.0)
    den = jnp.sum(ex, axis=1, keepdims=True)
    gw = ex / den
    gw_ref[...] = gw
    conf = jnp.max(gw, axis=1, keepdims=True)
    scale_ref[...] = (1.0 - conf) * sig_ref[0, 0]
    idxv = jnp.min(jnp.where(neglog == m, col, EP), axis=1, keepdims=True)
    oh = jnp.where(col == idxv, 1.0, 0.0)
    oh_ref[...] = oh
    counts = jnp.sum(oh, axis=0, keepdims=True)
    counts_ref[...] = counts
    gsum = jnp.sum(gw, axis=0, keepdims=True)
    aux_ref[...] = jnp.sum((counts / NT) * (gsum / NT), axis=1,
                           keepdims=True) * (E * 0.01)
    pc = jnp.ceil(counts * (1.0 / T)) * T
    iu = lax.broadcasted_iota(jnp.int32, (EP, EP), 0)
    ju = lax.broadcasted_iota(jnp.int32, (EP, EP), 1)
    su = (iu < ju).astype(jnp.float32)
    pad_off = jnp.dot(pc, su, preferred_element_type=jnp.float32)
    R = 128
    li = lax.broadcasted_iota(jnp.int32, (R, R), 0)
    lj = lax.broadcasted_iota(jnp.int32, (R, R), 1)
    ls = (lj < li).astype(jnp.float32)

    def body(c, prefix):
        ohc = oh_ref[pl.ds(c * R, R), :]
        ranks = jnp.dot(ls, ohc, preferred_element_type=jnp.float32)
        destc = jnp.sum((ranks + prefix + pad_off) * ohc, axis=1,
                        keepdims=True)
        dest_ref[pl.ds(c * R, R), :] = destc.astype(jnp.int32)
        return prefix + jnp.sum(ohc, axis=0, keepdims=True)

    lax.fori_loop(0, NT // R, body, jnp.zeros((1, EP), jnp.float32))


def _router(x2, wgp, bgp, sig):
    return pl.pallas_call(
        _router_body,
        out_shape=[
            jax.ShapeDtypeStruct((NT, EP), jnp.float32),
            jax.ShapeDtypeStruct((NT, 1), jnp.float32),
            jax.ShapeDtypeStruct((NT, 1), jnp.int32),
            jax.ShapeDtypeStruct((1, EP), jnp.float32),
            jax.ShapeDtypeStruct((1, 1), jnp.float32),
        ],
        scratch_shapes=[pltpu.VMEM((NT, EP), jnp.float32)],
    )(x2, wgp, bgp, sig)


def _sc_scatter(x2, dest):
    info = plsc.get_sparse_core_info()
    nc, ns = info.num_cores, info.num_subcores
    per_w = NT // (nc * ns)
    mesh = plsc.VectorSubcoreMesh(core_axis_name="c", subcore_axis_name="s")

    @functools.partial(
        pl.kernel, mesh=mesh,
        out_type=jax.ShapeDtypeStruct((NPAD, D), jnp.float32),
        scratch_types=[
            pltpu.VMEM((SUB,), jnp.int32),
            pltpu.VMEM((SUB, D), jnp.float32),
            pltpu.SemaphoreType.DMA,
        ],
    )
    def k(x_hbm, dest_hbm, out_hbm, idx_v, rows_v, sem):
        wid = lax.axis_index("s") * nc + lax.axis_index("c")
        for s in range(per_w // SUB):
            base = wid * per_w + s * SUB
            pltpu.sync_copy(dest_hbm.at[pl.ds(base, SUB)], idx_v)
            pltpu.sync_copy(x_hbm.at[pl.ds(base, SUB)], rows_v)
            pltpu.async_copy(rows_v, out_hbm.at[idx_v], sem).wait()

    return k(x2, dest)


def _sc_gather(src, dest):
    info = plsc.get_sparse_core_info()
    nc, ns = info.num_cores, info.num_subcores
    per_w = NT // (nc * ns)
    mesh = plsc.VectorSubcoreMesh(core_axis_name="c", subcore_axis_name="s")

    @functools.partial(
        pl.kernel, mesh=mesh,
        out_type=jax.ShapeDtypeStruct((NT, D), jnp.float32),
        scratch_types=[
            pltpu.VMEM((SUB,), jnp.int32),
            pltpu.VMEM((SUB, D), jnp.float32),
            pltpu.SemaphoreType.DMA,
        ],
    )
    def k(src_hbm, dest_hbm, out_hbm, idx_v, rows_v, sem):
        wid = lax.axis_index("s") * nc + lax.axis_index("c")
        for s in range(per_w // SUB):
            base = wid * per_w + s * SUB
            pltpu.sync_copy(dest_hbm.at[pl.ds(base, SUB)], idx_v)
            pltpu.async_copy(src_hbm.at[idx_v], rows_v, sem).wait()
            pltpu.sync_copy(rows_v, out_hbm.at[pl.ds(base, SUB)])

    return k(src, dest)


def _grouped_body(be_ref, x_ref, w1_ref, b1_ref, w2_ref, b2_ref, o_ref,
                  acc_ref):
    k = pl.program_id(1)
    h = jnp.dot(x_ref[...], w1_ref[0],
                preferred_element_type=jnp.float32) + b1_ref[0]
    part = jnp.dot(_gelu(h), w2_ref[0], preferred_element_type=jnp.float32)

    @pl.when(k == 0)
    def _():
        acc_ref[...] = part

    @pl.when(k != 0)
    def _():
        acc_ref[...] += part

    @pl.when(k == KH - 1)
    def _():
        o_ref[...] = acc_ref[...] + b2_ref[0]


def _grouped(be, x_s, W1, b1, W2, b2):
    grid_spec = pltpu.PrefetchScalarGridSpec(
        num_scalar_prefetch=1,
        grid=(NBR, KH),
        in_specs=[
            pl.BlockSpec((T, D), lambda b, k, be: (b, 0)),
            pl.BlockSpec((1, D, HT), lambda b, k, be: (be[b], 0, k)),
            pl.BlockSpec((1, 1, HT), lambda b, k, be: (be[b], 0, k)),
            pl.BlockSpec((1, HT, D), lambda b, k, be: (be[b], k, 0)),
            pl.BlockSpec((1, 1, D), lambda b, k, be: (be[b], 0, 0)),
        ],
        out_specs=pl.BlockSpec((T, D), lambda b, k, be: (b, 0)),
        scratch_shapes=[pltpu.VMEM((T, D), jnp.float32)],
    )
    return pl.pallas_call(
        _grouped_body,
        grid_spec=grid_spec,
        out_shape=jax.ShapeDtypeStruct((NPAD, D), jnp.float32),
    )(be, x_s, W1, b1.reshape(E, 1, H), W2, b2.reshape(E, 1, D))


def _shared_body(x_ref, w1_ref, b1_ref, w2_ref, b2_ref, r_ref, s_ref, o_ref,
                 acc_ref):
    k = pl.program_id(1)
    h = jnp.dot(x_ref[...], w1_ref[...],
                preferred_element_type=jnp.float32) + b1_ref[...]
    part = jnp.dot(_gelu(h), w2_ref[...], preferred_element_type=jnp.float32)

    @pl.when(k == 0)
    def _():
        acc_ref[...] = part

    @pl.when(k != 0)
    def _():
        acc_ref[...] += part

    @pl.when(k == KHS - 1)
    def _():
        o_ref[...] = r_ref[...] + s_ref[...] * (acc_ref[...] + b2_ref[...])


def _shared(x2, Ws1, bs1r, Ws2, bs2r, routed_tok, scale):
    nb = NT // TS
    return pl.pallas_call(
        _shared_body,
        grid=(nb, KHS),
        in_specs=[
            pl.BlockSpec((TS, D), lambda b, k: (b, 0)),
            pl.BlockSpec((D, HTS), lambda b, k: (0, k)),
            pl.BlockSpec((1, HTS), lambda b, k: (0, k)),
            pl.BlockSpec((HTS, D), lambda b, k: (k, 0)),
            pl.BlockSpec((1, D), lambda b, k: (0, 0)),
            pl.BlockSpec((TS, D), lambda b, k: (b, 0)),
            pl.BlockSpec((TS, 1), lambda b, k: (b, 0)),
        ],
        out_specs=pl.BlockSpec((TS, D), lambda b, k: (b, 0)),
        out_shape=jax.ShapeDtypeStruct((NT, D), jnp.float32),
        scratch_shapes=[pltpu.VMEM((TS, D), jnp.float32)],
    )(x2, Ws1, bs1r, Ws2, bs2r, routed_tok, scale)


def kernel(x, Wg, bg, W1, b1, W2, b2, Ws1, bs1, Ws2, bs2, shared_w):
    bb, ss, _ = x.shape
    x2 = x.reshape(NT, D)
    wgp = jnp.pad(Wg, ((0, 0), (0, EP - E)))
    bgp = jnp.pad(bg, (0, EP - E)).reshape(1, EP)
    sig = jax.nn.sigmoid(shared_w).reshape(1, 1)

    gw, scale, dest, counts, aux = _router(x2, wgp, bgp, sig)

    counts8 = counts[0, :E]
    pc = jnp.ceil(counts8 / T).astype(jnp.int32) * T
    off = jnp.concatenate([jnp.zeros((1,), jnp.int32),
                           jnp.cumsum(pc)[:-1]])
    be = (jnp.searchsorted(off, jnp.arange(NBR, dtype=jnp.int32) * T,
                           side="right") - 1).astype(jnp.int32)
    be = jnp.clip(be, 0, E - 1)

    dest_flat = dest.reshape(NT)
    x_sorted = _sc_scatter(x2, dest_flat)
    routed = _grouped(be, x_sorted, W1, b1, W2, b2)
    routed_tok = _sc_gather(routed, dest_flat)
    final2 = _shared(x2, Ws1, bs1.reshape(1, H), Ws2, bs2.reshape(1, D),
                     routed_tok, scale)

    return (final2.reshape(bb, ss, D), aux.reshape(()),
            gw[:, :E].reshape(bb, ss, E))


# speedup vs baseline: 1.0596x; 1.0596x over previous
import functools

import jax
import jax.numpy as jnp
from jax import lax
from jax.experimental import pallas as pl
from jax.experimental.pallas import tpu as pltpu
from jax.experimental.pallas import tpu_sc as plsc

D = 2048
H = 4096
E = 8
NT = 4096
EP = 128
T = 512
NBR = NT // T + E - 1
NPAD = NBR * T
HT = 1024
KH = H // HT
TS1 = 512
HT1 = 1024
KH1 = H // HT1
TS2 = 256
DT2 = 1024
JD2 = D // DT2
SUB = 32

_INV_SQRT2 = 0.7071067811865476


def _gelu(z):
    return 0.5 * z * (1.0 + lax.erf(z * _INV_SQRT2))


def _router_body(x_ref, wg_ref, bg_ref, sig_ref, gw_ref, scale_ref, dest_ref,
                 counts_ref, aux_ref, oh_ref):
    logits = jnp.dot(x_ref[...], wg_ref[...],
                     preferred_element_type=jnp.float32) + bg_ref[...]
    col = lax.broadcasted_iota(jnp.int32, (NT, EP), 1)
    valid = col < E
    neglog = jnp.where(valid, logits, -jnp.inf)
    m = jnp.max(neglog, axis=1, keepdims=True)
    ex = jnp.where(valid, jnp.exp(neglog - m), 0.0)
    den = jnp.sum(ex, axis=1, keepdims=True)
    gw = ex / den
    gw_ref[...] = gw
    conf = jnp.max(gw, axis=1, keepdims=True)
    scale_ref[...] = (1.0 - conf) * sig_ref[0, 0]
    idxv = jnp.min(jnp.where(neglog == m, col, EP), axis=1, keepdims=True)
    oh = jnp.where(col == idxv, 1.0, 0.0)
    oh_ref[...] = oh
    counts = jnp.sum(oh, axis=0, keepdims=True)
    counts_ref[...] = counts
    gsum = jnp.sum(gw, axis=0, keepdims=True)
    aux_ref[...] = jnp.sum((counts / NT) * (gsum / NT), axis=1,
                           keepdims=True) * (E * 0.01)
    pc = jnp.ceil(counts * (1.0 / T)) * T
    iu = lax.broadcasted_iota(jnp.int32, (EP, EP), 0)
    ju = lax.broadcasted_iota(jnp.int32, (EP, EP), 1)
    su = (iu < ju).astype(jnp.float32)
    pad_off = jnp.dot(pc, su, preferred_element_type=jnp.float32)
    R = 128
    li = lax.broadcasted_iota(jnp.int32, (R, R), 0)
    lj = lax.broadcasted_iota(jnp.int32, (R, R), 1)
    ls = (lj < li).astype(jnp.float32)

    def body(c, prefix):
        ohc = oh_ref[pl.ds(c * R, R), :]
        ranks = jnp.dot(ls, ohc, preferred_element_type=jnp.float32)
        destc = jnp.sum((ranks + prefix + pad_off) * ohc, axis=1,
                        keepdims=True)
        dest_ref[pl.ds(c * R, R), :] = destc.astype(jnp.int32)
        return prefix + jnp.sum(ohc, axis=0, keepdims=True)

    lax.fori_loop(0, NT // R, body, jnp.zeros((1, EP), jnp.float32))


def _router(x2, wgp, bgp, sig):
    return pl.pallas_call(
        _router_body,
        out_shape=[
            jax.ShapeDtypeStruct((NT, EP), jnp.float32),
            jax.ShapeDtypeStruct((NT, 1), jnp.float32),
            jax.ShapeDtypeStruct((NT, 1), jnp.int32),
            jax.ShapeDtypeStruct((1, EP), jnp.float32),
            jax.ShapeDtypeStruct((1, 1), jnp.float32),
        ],
        scratch_shapes=[pltpu.VMEM((NT, EP), jnp.float32)],
    )(x2, wgp, bgp, sig)


def _sc_scatter(x2, dest):
    info = plsc.get_sparse_core_info()
    nc, ns = info.num_cores, info.num_subcores
    per_w = NT // (nc * ns)
    mesh = plsc.VectorSubcoreMesh(core_axis_name="c", subcore_axis_name="s")

    @functools.partial(
        pl.kernel, mesh=mesh,
        out_type=jax.ShapeDtypeStruct((NPAD, D), jnp.float32),
        scratch_types=[
            pltpu.VMEM((SUB,), jnp.int32),
            pltpu.VMEM((SUB, D), jnp.float32),
            pltpu.SemaphoreType.DMA,
        ],
    )
    def k(x_hbm, dest_hbm, out_hbm, idx_v, rows_v, sem):
        wid = lax.axis_index("s") * nc + lax.axis_index("c")
        for s in range(per_w // SUB):
            base = wid * per_w + s * SUB
            pltpu.sync_copy(dest_hbm.at[pl.ds(base, SUB)], idx_v)
            pltpu.sync_copy(x_hbm.at[pl.ds(base, SUB)], rows_v)
            pltpu.async_copy(rows_v, out_hbm.at[idx_v], sem).wait()

    return k(x2, dest)


def _sc_gather(src, dest):
    info = plsc.get_sparse_core_info()
    nc, ns = info.num_cores, info.num_subcores
    per_w = NT // (nc * ns)
    mesh = plsc.VectorSubcoreMesh(core_axis_name="c", subcore_axis_name="s")

    @functools.partial(
        pl.kernel, mesh=mesh,
        out_type=jax.ShapeDtypeStruct((NT, D), jnp.float32),
        scratch_types=[
            pltpu.VMEM((SUB,), jnp.int32),
            pltpu.VMEM((SUB, D), jnp.float32),
            pltpu.SemaphoreType.DMA,
        ],
    )
    def k(src_hbm, dest_hbm, out_hbm, idx_v, rows_v, sem):
        wid = lax.axis_index("s") * nc + lax.axis_index("c")
        for s in range(per_w // SUB):
            base = wid * per_w + s * SUB
            pltpu.sync_copy(dest_hbm.at[pl.ds(base, SUB)], idx_v)
            pltpu.async_copy(src_hbm.at[idx_v], rows_v, sem).wait()
            pltpu.sync_copy(rows_v, out_hbm.at[pl.ds(base, SUB)])

    return k(src, dest)


def _grouped_body(be_ref, x_ref, w1_ref, b1_ref, w2_ref, b2_ref, o_ref,
                  acc_ref):
    k = pl.program_id(1)
    h = jnp.dot(x_ref[...], w1_ref[0],
                preferred_element_type=jnp.float32) + b1_ref[0]
    part = jnp.dot(_gelu(h), w2_ref[0], preferred_element_type=jnp.float32)

    @pl.when(k == 0)
    def _():
        acc_ref[...] = part

    @pl.when(k != 0)
    def _():
        acc_ref[...] += part

    @pl.when(k == KH - 1)
    def _():
        o_ref[...] = acc_ref[...] + b2_ref[0]


def _grouped(be, x_s, W1, b1, W2, b2):
    grid_spec = pltpu.PrefetchScalarGridSpec(
        num_scalar_prefetch=1,
        grid=(NBR, KH),
        in_specs=[
            pl.BlockSpec((T, D), lambda b, k, be: (b, 0)),
            pl.BlockSpec((1, D, HT), lambda b, k, be: (be[b], 0, k)),
            pl.BlockSpec((1, 1, HT), lambda b, k, be: (be[b], 0, k)),
            pl.BlockSpec((1, HT, D), lambda b, k, be: (be[b], k, 0)),
            pl.BlockSpec((1, 1, D), lambda b, k, be: (be[b], 0, 0)),
        ],
        out_specs=pl.BlockSpec((T, D), lambda b, k, be: (b, 0)),
        scratch_shapes=[pltpu.VMEM((T, D), jnp.float32)],
    )
    return pl.pallas_call(
        _grouped_body,
        grid_spec=grid_spec,
        out_shape=jax.ShapeDtypeStruct((NPAD, D), jnp.float32),
    )(be, x_s, W1, b1.reshape(E, 1, H), W2, b2.reshape(E, 1, D))


def _shared1_body(x_ref, w1_ref, b1_ref, h_ref):
    h = jnp.dot(x_ref[...], w1_ref[...],
                preferred_element_type=jnp.float32) + b1_ref[...]
    h_ref[...] = _gelu(h)


def _shared1(x2, Ws1, bs1r):
    nb = NT // TS1
    return pl.pallas_call(
        _shared1_body,
        grid=(KH1, nb),
        in_specs=[
            pl.BlockSpec((TS1, D), lambda k, b: (b, 0)),
            pl.BlockSpec((D, HT1), lambda k, b: (0, k)),
            pl.BlockSpec((1, HT1), lambda k, b: (0, k)),
        ],
        out_specs=pl.BlockSpec((TS1, HT1), lambda k, b: (b, k)),
        out_shape=jax.ShapeDtypeStruct((NT, H), jnp.float32),
    )(x2, Ws1, bs1r)


def _shared2_body(h_ref, w2_ref, b2_ref, r_ref, s_ref, o_ref):
    sh = jnp.dot(h_ref[...], w2_ref[...],
                 preferred_element_type=jnp.float32) + b2_ref[...]
    o_ref[...] = r_ref[...] + s_ref[...] * sh


def _shared2(h_all, Ws2, bs2r, routed_tok, scale):
    nb = NT // TS2
    return pl.pallas_call(
        _shared2_body,
        grid=(JD2, nb),
        in_specs=[
            pl.BlockSpec((TS2, H), lambda j, b: (b, 0)),
            pl.BlockSpec((H, DT2), lambda j, b: (0, j)),
            pl.BlockSpec((1, DT2), lambda j, b: (0, j)),
            pl.BlockSpec((TS2, DT2), lambda j, b: (b, j)),
            pl.BlockSpec((TS2, 1), lambda j, b: (b, 0)),
        ],
        out_specs=pl.BlockSpec((TS2, DT2), lambda j, b: (b, j)),
        out_shape=jax.ShapeDtypeStruct((NT, D), jnp.float32),
    )(h_all, Ws2, bs2r, routed_tok, scale)


def kernel(x, Wg, bg, W1, b1, W2, b2, Ws1, bs1, Ws2, bs2, shared_w):
    bb, ss, _ = x.shape
    x2 = x.reshape(NT, D)
    wgp = jnp.pad(Wg, ((0, 0), (0, EP - E)))
    bgp = jnp.pad(bg, (0, EP - E)).reshape(1, EP)
    sig = jax.nn.sigmoid(shared_w).reshape(1, 1)

    gw, scale, dest, counts, aux = _router(x2, wgp, bgp, sig)

    counts8 = counts[0, :E]
    pc = jnp.ceil(counts8 / T).astype(jnp.int32) * T
    off = jnp.concatenate([jnp.zeros((1,), jnp.int32),
                           jnp.cumsum(pc)[:-1]])
    be = (jnp.searchsorted(off, jnp.arange(NBR, dtype=jnp.int32) * T,
                           side="right") - 1).astype(jnp.int32)
    be = jnp.clip(be, 0, E - 1)

    dest_flat = dest.reshape(NT)
    x_sorted = _sc_scatter(x2, dest_flat)
    routed = _grouped(be, x_sorted, W1, b1, W2, b2)
    routed_tok = _sc_gather(routed, dest_flat)
    hs = _shared1(x2, Ws1, bs1.reshape(1, H))
    final2 = _shared2(hs, Ws2, bs2.reshape(1, D), routed_tok, scale)

    return (final2.reshape(bb, ss, D), aux.reshape(()),
            gw[:, :E].reshape(bb, ss, E))


# speedup vs baseline: 1.1066x; 1.0443x over previous
import functools

import jax
import jax.numpy as jnp
from jax import lax
from jax.experimental import pallas as pl
from jax.experimental.pallas import tpu as pltpu
from jax.experimental.pallas import tpu_sc as plsc

D = 2048
H = 4096
E = 8
NT = 4096
EP = 128
T = 512
NBR = NT // T + E - 1
NPAD = NBR * T
HT = 1024
KH = H // HT
TS1 = 512
HT1 = 2048
KH1 = H // HT1
TS2 = 512
DT2 = 1024
JD2 = D // DT2
SUB = 32

_INV_SQRT2 = 0.7071067811865476


def _gelu(z):
    return 0.5 * z * (1.0 + lax.erf(z * _INV_SQRT2))


def _router_body(x_ref, wg_ref, bg_ref, sig_ref, gw_ref, scale_ref, dest_ref,
                 counts_ref, aux_ref, oh_ref):
    logits = jnp.dot(x_ref[...], wg_ref[...],
                     preferred_element_type=jnp.float32) + bg_ref[...]
    col = lax.broadcasted_iota(jnp.int32, (NT, EP), 1)
    valid = col < E
    neglog = jnp.where(valid, logits, -jnp.inf)
    m = jnp.max(neglog, axis=1, keepdims=True)
    ex = jnp.where(valid, jnp.exp(neglog - m), 0.0)
    den = jnp.sum(ex, axis=1, keepdims=True)
    gw = ex / den
    gw_ref[...] = gw
    conf = jnp.max(gw, axis=1, keepdims=True)
    scale_ref[...] = (1.0 - conf) * sig_ref[0, 0]
    idxv = jnp.min(jnp.where(neglog == m, col, EP), axis=1, keepdims=True)
    oh = jnp.where(col == idxv, 1.0, 0.0)
    oh_ref[...] = oh
    counts = jnp.sum(oh, axis=0, keepdims=True)
    counts_ref[...] = counts
    gsum = jnp.sum(gw, axis=0, keepdims=True)
    aux_ref[...] = jnp.sum((counts / NT) * (gsum / NT), axis=1,
                           keepdims=True) * (E * 0.01)
    pc = jnp.ceil(counts * (1.0 / T)) * T
    iu = lax.broadcasted_iota(jnp.int32, (EP, EP), 0)
    ju = lax.broadcasted_iota(jnp.int32, (EP, EP), 1)
    su = (iu < ju).astype(jnp.float32)
    pad_off = jnp.dot(pc, su, preferred_element_type=jnp.float32)
    R = 128
    li = lax.broadcasted_iota(jnp.int32, (R, R), 0)
    lj = lax.broadcasted_iota(jnp.int32, (R, R), 1)
    ls = (lj < li).astype(jnp.float32)

    def body(c, prefix):
        ohc = oh_ref[pl.ds(c * R, R), :]
        ranks = jnp.dot(ls, ohc, preferred_element_type=jnp.float32)
        destc = jnp.sum((ranks + prefix + pad_off) * ohc, axis=1,
                        keepdims=True)
        dest_ref[pl.ds(c * R, R), :] = destc.astype(jnp.int32)
        return prefix + jnp.sum(ohc, axis=0, keepdims=True)

    lax.fori_loop(0, NT // R, body, jnp.zeros((1, EP), jnp.float32))


def _router(x2, wgp, bgp, sig):
    return pl.pallas_call(
        _router_body,
        out_shape=[
            jax.ShapeDtypeStruct((NT, EP), jnp.float32),
            jax.ShapeDtypeStruct((NT, 1), jnp.float32),
            jax.ShapeDtypeStruct((NT, 1), jnp.int32),
            jax.ShapeDtypeStruct((1, EP), jnp.float32),
            jax.ShapeDtypeStruct((1, 1), jnp.float32),
        ],
        scratch_shapes=[pltpu.VMEM((NT, EP), jnp.float32)],
    )(x2, wgp, bgp, sig)


def _sc_scatter(x2, dest):
    info = plsc.get_sparse_core_info()
    nc, ns = info.num_cores, info.num_subcores
    per_w = NT // (nc * ns)
    mesh = plsc.VectorSubcoreMesh(core_axis_name="c", subcore_axis_name="s")

    @functools.partial(
        pl.kernel, mesh=mesh,
        out_type=jax.ShapeDtypeStruct((NPAD, D), jnp.float32),
        scratch_types=[
            pltpu.VMEM((SUB,), jnp.int32),
            pltpu.VMEM((SUB, D), jnp.float32),
            pltpu.SemaphoreType.DMA,
        ],
    )
    def k(x_hbm, dest_hbm, out_hbm, idx_v, rows_v, sem):
        wid = lax.axis_index("s") * nc + lax.axis_index("c")
        for s in range(per_w // SUB):
            base = wid * per_w + s * SUB
            pltpu.sync_copy(dest_hbm.at[pl.ds(base, SUB)], idx_v)
            pltpu.sync_copy(x_hbm.at[pl.ds(base, SUB)], rows_v)
            pltpu.async_copy(rows_v, out_hbm.at[idx_v], sem).wait()

    return k(x2, dest)


def _sc_gather(src, dest):
    info = plsc.get_sparse_core_info()
    nc, ns = info.num_cores, info.num_subcores
    per_w = NT // (nc * ns)
    mesh = plsc.VectorSubcoreMesh(core_axis_name="c", subcore_axis_name="s")

    @functools.partial(
        pl.kernel, mesh=mesh,
        out_type=jax.ShapeDtypeStruct((NT, D), jnp.float32),
        scratch_types=[
            pltpu.VMEM((SUB,), jnp.int32),
            pltpu.VMEM((SUB, D), jnp.float32),
            pltpu.SemaphoreType.DMA,
        ],
    )
    def k(src_hbm, dest_hbm, out_hbm, idx_v, rows_v, sem):
        wid = lax.axis_index("s") * nc + lax.axis_index("c")
        for s in range(per_w // SUB):
            base = wid * per_w + s * SUB
            pltpu.sync_copy(dest_hbm.at[pl.ds(base, SUB)], idx_v)
            pltpu.async_copy(src_hbm.at[idx_v], rows_v, sem).wait()
            pltpu.sync_copy(rows_v, out_hbm.at[pl.ds(base, SUB)])

    return k(src, dest)


def _grouped_body(be_ref, x_ref, w1_ref, b1_ref, w2_ref, b2_ref, o_ref,
                  acc_ref):
    k = pl.program_id(1)
    h = jnp.dot(x_ref[...], w1_ref[0],
                preferred_element_type=jnp.float32) + b1_ref[0]
    part = jnp.dot(_gelu(h), w2_ref[0], preferred_element_type=jnp.float32)

    @pl.when(k == 0)
    def _():
        acc_ref[...] = part

    @pl.when(k != 0)
    def _():
        acc_ref[...] += part

    @pl.when(k == KH - 1)
    def _():
        o_ref[...] = acc_ref[...] + b2_ref[0]


def _grouped(be, x_s, W1, b1, W2, b2):
    grid_spec = pltpu.PrefetchScalarGridSpec(
        num_scalar_prefetch=1,
        grid=(NBR, KH),
        in_specs=[
            pl.BlockSpec((T, D), lambda b, k, be: (b, 0)),
            pl.BlockSpec((1, D, HT), lambda b, k, be: (be[b], 0, k)),
            pl.BlockSpec((1, 1, HT), lambda b, k, be: (be[b], 0, k)),
            pl.BlockSpec((1, HT, D), lambda b, k, be: (be[b], k, 0)),
            pl.BlockSpec((1, 1, D), lambda b, k, be: (be[b], 0, 0)),
        ],
        out_specs=pl.BlockSpec((T, D), lambda b, k, be: (b, 0)),
        scratch_shapes=[pltpu.VMEM((T, D), jnp.float32)],
    )
    return pl.pallas_call(
        _grouped_body,
        grid_spec=grid_spec,
        out_shape=jax.ShapeDtypeStruct((NPAD, D), jnp.float32),
    )(be, x_s, W1, b1.reshape(E, 1, H), W2, b2.reshape(E, 1, D))


def _shared1_body(x_ref, w1_ref, b1_ref, h_ref):
    h = jnp.dot(x_ref[...], w1_ref[...],
                preferred_element_type=jnp.float32) + b1_ref[...]
    h_ref[...] = _gelu(h)


def _shared1(x2, Ws1, bs1r):
    nb = NT // TS1
    return pl.pallas_call(
        _shared1_body,
        grid=(KH1, nb),
        in_specs=[
            pl.BlockSpec((TS1, D), lambda k, b: (b, 0)),
            pl.BlockSpec((D, HT1), lambda k, b: (0, k)),
            pl.BlockSpec((1, HT1), lambda k, b: (0, k)),
        ],
        out_specs=pl.BlockSpec((TS1, HT1), lambda k, b: (b, k)),
        out_shape=jax.ShapeDtypeStruct((NT, H), jnp.float32),
    )(x2, Ws1, bs1r)


def _shared2_body(h_ref, w2_ref, b2_ref, r_ref, s_ref, o_ref):
    sh = jnp.dot(h_ref[...], w2_ref[...],
                 preferred_element_type=jnp.float32) + b2_ref[...]
    o_ref[...] = r_ref[...] + s_ref[...] * sh


def _shared2(h_all, Ws2, bs2r, routed_tok, scale):
    nb = NT // TS2
    return pl.pallas_call(
        _shared2_body,
        grid=(JD2, nb),
        in_specs=[
            pl.BlockSpec((TS2, H), lambda j, b: (b, 0)),
            pl.BlockSpec((H, DT2), lambda j, b: (0, j)),
            pl.BlockSpec((1, DT2), lambda j, b: (0, j)),
            pl.BlockSpec((TS2, DT2), lambda j, b: (b, j)),
            pl.BlockSpec((TS2, 1), lambda j, b: (b, 0)),
        ],
        out_specs=pl.BlockSpec((TS2, DT2), lambda j, b: (b, j)),
        out_shape=jax.ShapeDtypeStruct((NT, D), jnp.float32),
    )(h_all, Ws2, bs2r, routed_tok, scale)


def kernel(x, Wg, bg, W1, b1, W2, b2, Ws1, bs1, Ws2, bs2, shared_w):
    bb, ss, _ = x.shape
    x2 = x.reshape(NT, D)
    wgp = jnp.pad(Wg, ((0, 0), (0, EP - E)))
    bgp = jnp.pad(bg, (0, EP - E)).reshape(1, EP)
    sig = jax.nn.sigmoid(shared_w).reshape(1, 1)

    gw, scale, dest, counts, aux = _router(x2, wgp, bgp, sig)

    counts8 = counts[0, :E]
    pc = jnp.ceil(counts8 / T).astype(jnp.int32) * T
    off = jnp.concatenate([jnp.zeros((1,), jnp.int32),
                           jnp.cumsum(pc)[:-1]])
    be = (jnp.searchsorted(off, jnp.arange(NBR, dtype=jnp.int32) * T,
                           side="right") - 1).astype(jnp.int32)
    be = jnp.clip(be, 0, E - 1)

    dest_flat = dest.reshape(NT)
    x_sorted = _sc_scatter(x2, dest_flat)
    routed = _grouped(be, x_sorted, W1, b1, W2, b2)
    routed_tok = _sc_gather(routed, dest_flat)
    hs = _shared1(x2, Ws1, bs1.reshape(1, H))
    final2 = _shared2(hs, Ws2, bs2.reshape(1, D), routed_tok, scale)

    return (final2.reshape(bb, ss, D), aux.reshape(()),
            gw[:, :E].reshape(bb, ss, E))
